# Initial kernel scaffold; baseline (speedup 1.0000x reference)
import jax
import jax.numpy as jnp
from jax.experimental import pallas as pl


def kernel(vectors, impl_sets, table, W):
    raise NotImplementedError("write your pallas kernel here")


# speedup vs baseline: 3.4300x; 3.4300x over previous
import dataclasses
import functools

import jax
import jax.numpy as jnp
from jax import lax
from jax.experimental import pallas as pl
from jax.experimental.pallas import tpu as pltpu
from jax.experimental.pallas import tpu_sc as plsc

B = 1024
K = 512
D = 128
LANES = 16
NW = 32
B_PER_W = B // NW
KCH = 128
NKCH = K // KCH
NCH = D // LANES


def _q_body(v_ref, wt_ref, q_ref):
    q_ref[...] = jnp.dot(v_ref[...], wt_ref[...],
                         preferred_element_type=jnp.float32)


def _sc_scores(q, impl_sets3, table):
    mesh = plsc.VectorSubcoreMesh(core_axis_name="c", subcore_axis_name="s")
    cp = pltpu.CompilerParams()
    if "needs_layout_passes" in pltpu.CompilerParams.__dataclass_fields__:
        cp = dataclasses.replace(cp, needs_layout_passes=False)

    @functools.partial(
        pl.kernel,
        out_type=jax.ShapeDtypeStruct((B, K), jnp.float32),
        mesh=mesh,
        compiler_params=cp,
        scratch_types=[
            pltpu.VMEM((NKCH, KCH), jnp.int32),
            pltpu.VMEM((K, D), jnp.float32),
            pltpu.VMEM((D,), jnp.float32),
            pltpu.VMEM((K,), jnp.float32),
            pltpu.VMEM((LANES, LANES + 1), jnp.float32),
        ],
    )
    def k(q_hbm, idx_hbm, table_hbm, out_hbm, idx_v, rows_v, q_v, s_v, p_v):
        wid = lax.axis_index("s") * 2 + lax.axis_index("c")
        row_ids = lax.iota(jnp.int32, LANES)

        @pl.loop(0, B_PER_W)
        def _(bi):
            b = wid * B_PER_W + bi
            pltpu.sync_copy(idx_hbm.at[b], idx_v)
            pltpu.sync_copy(q_hbm.at[b], q_v)
            for j in range(NKCH):
                pltpu.sync_copy(table_hbm.at[idx_v.at[j]],
                                rows_v.at[pl.ds(j * KCH, KCH)])
            qs = [q_v[pl.ds(c * LANES, LANES)] for c in range(NCH)]

            @pl.loop(0, K, step=LANES)
            def _(r0):
                for j in range(LANES):
                    acc = rows_v[r0 + j, pl.ds(0, LANES)] * qs[0]
                    for c in range(1, NCH):
                        acc = acc + rows_v[r0 + j, pl.ds(c * LANES, LANES)] * qs[c]
                    p_v[j, pl.ds(0, LANES)] = acc
                tot = plsc.load_gather(
                    p_v, [row_ids, jnp.zeros((LANES,), jnp.int32)])
                for l in range(1, LANES):
                    tot = tot + plsc.load_gather(
                        p_v, [row_ids, jnp.full((LANES,), l, jnp.int32)])
                s_v[pl.ds(r0, LANES)] = tot

            pltpu.sync_copy(s_v, out_hbm.at[b])

    return k(q, impl_sets3, table)


def kernel(vectors, impl_sets, table, W):
    q = pl.pallas_call(
        _q_body,
        out_shape=jax.ShapeDtypeStruct((B, D), jnp.float32),
    )(vectors, W.T)
    scores = _sc_scores(q, impl_sets.reshape(B, NKCH, KCH), table)
    return (impl_sets, scores)


# speedup vs baseline: 5.3757x; 1.5673x over previous
import dataclasses
import functools

import jax
import jax.numpy as jnp
from jax import lax
from jax.experimental import pallas as pl
from jax.experimental.pallas import tpu as pltpu
from jax.experimental.pallas import tpu_sc as plsc

B = 1024
K = 512
D = 128
LANES = 16
NW = 32
B_PER_W = B // NW
KCH = 128
NKCH = K // KCH
NCH = D // LANES


def _q_body(v_ref, wt_ref, q_ref):
    q_ref[...] = jnp.dot(v_ref[...], wt_ref[...],
                         preferred_element_type=jnp.float32)


NCHUNKS = B_PER_W * NKCH
NBUF = 2


def _sc_scores(q, impl_sets3, table):
    mesh = plsc.VectorSubcoreMesh(core_axis_name="c", subcore_axis_name="s")
    cp = pltpu.CompilerParams()
    if "needs_layout_passes" in pltpu.CompilerParams.__dataclass_fields__:
        cp = dataclasses.replace(cp, needs_layout_passes=False)

    @functools.partial(
        pl.kernel,
        out_type=jax.ShapeDtypeStruct((B, K), jnp.float32),
        mesh=mesh,
        compiler_params=cp,
        scratch_types=[
            pltpu.VMEM((NCHUNKS, KCH), jnp.int32),
            pltpu.VMEM((B_PER_W, D), jnp.float32),
            pltpu.VMEM((B_PER_W, K), jnp.float32),
            pltpu.VMEM((LANES, LANES + 1), jnp.float32),
        ]
        + [pltpu.VMEM((KCH, D), jnp.float32) for _ in range(NBUF)]
        + [pltpu.SemaphoreType.DMA for _ in range(NBUF)],
    )
    def k(q_hbm, idx_hbm, table_hbm, out_hbm, idx_v, q_v, s_v, p_v, *bufs_sems):
        bufs = bufs_sems[:NBUF]
        sems = bufs_sems[NBUF:]
        wid = lax.axis_index("s") * 2 + lax.axis_index("c")
        row_ids = lax.iota(jnp.int32, LANES)

        pltpu.sync_copy(idx_hbm.at[wid], idx_v)
        pltpu.sync_copy(q_hbm.at[pl.ds(wid * B_PER_W, B_PER_W)], q_v)

        def start(t, i):
            pltpu.async_copy(table_hbm.at[idx_v.at[t]], bufs[i], sems[i])

        def wait(i):
            pltpu.make_async_copy(
                table_hbm.at[pl.ds(0, KCH)], bufs[i], sems[i]).wait()

        for i in range(NBUF):
            start(i, i)

        @pl.loop(0, NCHUNKS, step=NBUF)
        def _(t0):
            for i in range(NBUF):
                t = t0 + i
                bl = t // NKCH
                col0 = (t % NKCH) * KCH
                wait(i)
                qs = [q_v[bl, pl.ds(c * LANES, LANES)] for c in range(NCH)]

                @pl.loop(0, KCH, step=LANES)
                def _(r0):
                    for j in range(LANES):
                        acc = bufs[i][r0 + j, pl.ds(0, LANES)] * qs[0]
                        for c in range(1, NCH):
                            acc = acc + (bufs[i][r0 + j, pl.ds(c * LANES, LANES)]
                                         * qs[c])
                        p_v[j, pl.ds(0, LANES)] = acc
                    tot = plsc.load_gather(
                        p_v, [row_ids, jnp.zeros((LANES,), jnp.int32)])
                    for l in range(1, LANES):
                        tot = tot + plsc.load_gather(
                            p_v, [row_ids, jnp.full((LANES,), l, jnp.int32)])
                    s_v[bl, pl.ds(col0 + r0, LANES)] = tot

                @pl.when(t + NBUF < NCHUNKS)
                def _():
                    start(t + NBUF, i)

        pltpu.sync_copy(s_v, out_hbm.at[pl.ds(wid * B_PER_W, B_PER_W)])

    return k(q, impl_sets3, table)


def kernel(vectors, impl_sets, table, W):
    q = pl.pallas_call(
        _q_body,
        out_shape=jax.ShapeDtypeStruct((B, D), jnp.float32),
    )(vectors, W.T)
    scores = _sc_scores(q, impl_sets.reshape(NW, NCHUNKS, KCH), table)
    return (impl_sets, scores)


# speedup vs baseline: 8.1088x; 1.5084x over previous
import dataclasses
import functools

import jax
import jax.numpy as jnp
from jax import lax
from jax.experimental import pallas as pl
from jax.experimental.pallas import tpu as pltpu
from jax.experimental.pallas import tpu_sc as plsc

B = 1024
K = 512
D = 128
LANES = 16
NW = 32
B_PER_W = B // NW
KCH = 128
NKCH = K // KCH
NCH = D // LANES


def _q_body(v_ref, wt_ref, q_ref):
    q_ref[...] = jnp.dot(v_ref[...], wt_ref[...],
                         preferred_element_type=jnp.float32)


NCHUNKS = B_PER_W * NKCH
NBUF = 2


def _sc_scores(q, impl_sets3, table):
    mesh = plsc.VectorSubcoreMesh(core_axis_name="c", subcore_axis_name="s")
    cp = pltpu.CompilerParams()
    if "needs_layout_passes" in pltpu.CompilerParams.__dataclass_fields__:
        cp = dataclasses.replace(cp, needs_layout_passes=False)

    @functools.partial(
        pl.kernel,
        out_type=jax.ShapeDtypeStruct((B, K), jnp.float32),
        mesh=mesh,
        compiler_params=cp,
        scratch_types=[
            pltpu.VMEM((NCHUNKS, KCH), jnp.int32),
            pltpu.VMEM((B_PER_W, D), jnp.float32),
            pltpu.VMEM((B_PER_W, K), jnp.float32),
            pltpu.VMEM((2, LANES, LANES + 1), jnp.float32),
        ]
        + [pltpu.VMEM((KCH, D), jnp.float32) for _ in range(NBUF)]
        + [pltpu.SemaphoreType.DMA for _ in range(NBUF)],
    )
    def k(q_hbm, idx_hbm, table_hbm, out_hbm, idx_v, q_v, s_v, p_v, *bufs_sems):
        bufs = bufs_sems[:NBUF]
        sems = bufs_sems[NBUF:]
        wid = lax.axis_index("s") * 2 + lax.axis_index("c")
        row_ids = lax.iota(jnp.int32, LANES)

        pltpu.sync_copy(idx_hbm.at[wid], idx_v)
        pltpu.sync_copy(q_hbm.at[pl.ds(wid * B_PER_W, B_PER_W)], q_v)

        def start(t, i):
            pltpu.async_copy(table_hbm.at[idx_v.at[t]], bufs[i], sems[i])

        def wait(i):
            pltpu.make_async_copy(
                table_hbm.at[pl.ds(0, KCH)], bufs[i], sems[i]).wait()

        for i in range(NBUF):
            start(i, i)

        @pl.loop(0, NCHUNKS, step=NBUF)
        def _(t0):
            for i in range(NBUF):
                t = t0 + i
                bl = t // NKCH
                col0 = (t % NKCH) * KCH
                wait(i)
                qs = [q_v[bl, pl.ds(c * LANES, LANES)] for c in range(NCH)]

                def group(r0, bank):
                    accs = [bufs[i][r0 + j, pl.ds(0, LANES)] * qs[0]
                            for j in range(LANES)]
                    for c in range(1, NCH):
                        for j in range(LANES):
                            accs[j] = accs[j] + (
                                bufs[i][r0 + j, pl.ds(c * LANES, LANES)] * qs[c])
                    for j in range(LANES):
                        p_v[bank, j, pl.ds(0, LANES)] = accs[j]
                    tot = plsc.load_gather(
                        p_v.at[bank],
                        [row_ids, jnp.zeros((LANES,), jnp.int32)])
                    for l in range(1, LANES):
                        tot = tot + plsc.load_gather(
                            p_v.at[bank],
                            [row_ids, jnp.full((LANES,), l, jnp.int32)])
                    s_v[bl, pl.ds(col0 + r0, LANES)] = tot

                @pl.loop(0, KCH, step=2 * LANES)
                def _(r0):
                    group(r0, 0)
                    group(r0 + LANES, 1)

                @pl.when(t + NBUF < NCHUNKS)
                def _():
                    start(t + NBUF, i)

        pltpu.sync_copy(s_v, out_hbm.at[pl.ds(wid * B_PER_W, B_PER_W)])

    return k(q, impl_sets3, table)


def kernel(vectors, impl_sets, table, W):
    q = pl.pallas_call(
        _q_body,
        out_shape=jax.ShapeDtypeStruct((B, D), jnp.float32),
    )(vectors, W.T)
    scores = _sc_scores(q, impl_sets.reshape(NW, NCHUNKS, KCH), table)
    return (impl_sets, scores)
